# baseline (speedup 1.0000x reference)
import functools

import jax
import jax.numpy as jnp
from jax import lax
from jax.experimental import pallas as pl
from jax.experimental.pallas import tpu as pltpu
from jax.experimental.pallas import tpu_sc as plsc

D = 32
V = 1000000
B = 4096
T = 200
NW = 32
LANES = 128

_mesh = plsc.VectorSubcoreMesh(core_axis_name="c", subcore_axis_name="s")


@functools.partial(
    pl.kernel,
    mesh=_mesh,
    compiler_params=pltpu.CompilerParams(
        use_tc_tiling_on_sc=False, needs_layout_passes=False),
    out_type=jax.ShapeDtypeStruct((T, 4, NW, 8 * LANES), jnp.float32),
    scratch_types=[
        pltpu.VMEM((LANES * T,), jnp.int32),
        pltpu.VMEM((2, LANES), jnp.int32),
        pltpu.VMEM((2, LANES, D), jnp.float32),
        pltpu.VMEM((2, 4, 8 * LANES), jnp.float32),
        pltpu.SemaphoreType.DMA((2,)),
        pltpu.SemaphoreType.DMA((2,)),
    ],
)
def _gather(idx_hbm, tab_hbm, out_hbm, idx_v, idxc, rows, otile, sem_g, sem_o):
    wid = lax.axis_index("s") * 2 + lax.axis_index("c")
    iota = lax.iota(jnp.int32, 16)
    pltpu.sync_copy(idx_hbm.at[pl.ds(wid * (LANES * T), LANES * T)], idx_v)

    def build_idxc(t, bb):
        for g in range(LANES // 16):
            base = T * 16 * g + T * iota
            idxc[bb, pl.ds(16 * g, 16)] = 4 * plsc.load_gather(
                idx_v, [base + t])

    def gather_desc(bb):
        return pltpu.make_async_copy(
            tab_hbm.at[idxc.at[bb]], rows.at[bb], sem_g.at[bb])

    def out_desc(t, bb):
        return pltpu.make_async_copy(
            otile.at[bb], out_hbm.at[t, :, wid], sem_o.at[bb])

    def transpose_rows(bb):
        @plsc.parallel_loop(0, D, unroll=8)
        def _tp(d):
            s = lax.div(d, 8)
            base = lax.rem(d, 8) * LANES
            dsplat = lax.broadcast(d, (16,))
            for gl in range(LANES // 16):
                vals = plsc.load_gather(
                    rows.at[bb], [16 * gl + iota, dsplat])
                otile[bb, s, pl.ds(base + 16 * gl, 16)] = vals

    build_idxc(0, 0)
    gather_desc(0).start()

    @pl.loop(0, T, step=2)
    def _steps(to):
        for bb in range(2):
            t = to + bb

            @pl.when(t + 1 < T)
            def _prefetch():
                build_idxc(t + 1, 1 - bb)
                @pl.when(t >= 1)
                def _():
                    out_desc(t - 1, 1 - bb).wait()
                gather_desc(1 - bb).start()

            gather_desc(bb).wait()
            transpose_rows(bb)
            out_desc(t, bb).start()

    out_desc(T - 2, 0).wait()
    out_desc(T - 1, 1).wait()


def kernel(x, table):
    idx = x.reshape(-1)
    tab_lin = jnp.pad(table, ((0, 0), (0, LANES - D))).reshape(4 * V, D)
    out4 = _gather(idx, tab_lin)
    out5 = out4.reshape(T, 4, NW, 8, LANES)
    return out5.transpose(2, 4, 0, 1, 3).reshape(B, T, D)


# speedup vs baseline: 1.0197x; 1.0197x over previous
import functools

import jax
import jax.numpy as jnp
from jax import lax
from jax.experimental import pallas as pl
from jax.experimental.pallas import tpu as pltpu
from jax.experimental.pallas import tpu_sc as plsc

D = 32
V = 1000000
B = 4096
T = 200
NW = 32
LANES = 128

_mesh = plsc.VectorSubcoreMesh(core_axis_name="c", subcore_axis_name="s")


@functools.partial(
    pl.kernel,
    mesh=_mesh,
    compiler_params=pltpu.CompilerParams(
        use_tc_tiling_on_sc=False, needs_layout_passes=False),
    out_type=jax.ShapeDtypeStruct((T, 4, NW, 8 * LANES), jnp.float32),
    scratch_types=[
        pltpu.VMEM((LANES * T,), jnp.int32),
        pltpu.VMEM((2, LANES), jnp.int32),
        pltpu.VMEM((2, LANES, D), jnp.float32),
        pltpu.VMEM((2, 4, 8 * LANES), jnp.float32),
        pltpu.SemaphoreType.DMA((2,)),
        pltpu.SemaphoreType.DMA((2,)),
    ],
)
def _gather(idx_hbm, tab_hbm, out_hbm, idx_v, idxc, rows, otile, sem_g, sem_o):
    wid = lax.axis_index("s") * 2 + lax.axis_index("c")
    iota = lax.iota(jnp.int32, 16)
    pltpu.sync_copy(idx_hbm.at[pl.ds(wid * (LANES * T), LANES * T)], idx_v)

    def build_idxc(t, bb):
        for g in range(LANES // 16):
            base = T * 16 * g + T * iota
            idxc[bb, pl.ds(16 * g, 16)] = 4 * plsc.load_gather(
                idx_v, [base + t])

    def gather_desc(bb):
        return pltpu.make_async_copy(
            tab_hbm.at[idxc.at[bb]], rows.at[bb], sem_g.at[bb])

    def out_desc(t, bb):
        return pltpu.make_async_copy(
            otile.at[bb], out_hbm.at[t, :, wid], sem_o.at[bb])

    def transpose_rows(bb):
        @plsc.parallel_loop(0, D, unroll=4)
        def _tp(d):
            s = lax.div(d, 8)
            base = lax.rem(d, 8) * LANES
            dsplat = lax.broadcast(d, (16,))
            for gl in range(LANES // 16):
                vals = plsc.load_gather(
                    rows.at[bb], [16 * gl + iota, dsplat])
                otile[bb, s, pl.ds(base + 16 * gl, 16)] = vals

    build_idxc(0, 0)
    gather_desc(0).start()

    @pl.loop(0, T, step=2)
    def _steps(to):
        for bb in range(2):
            t = to + bb

            @pl.when(t + 1 < T)
            def _prefetch():
                build_idxc(t + 1, 1 - bb)
                @pl.when(t >= 1)
                def _():
                    out_desc(t - 1, 1 - bb).wait()
                gather_desc(1 - bb).start()

            gather_desc(bb).wait()
            transpose_rows(bb)
            out_desc(t, bb).start()

    out_desc(T - 2, 0).wait()
    out_desc(T - 1, 1).wait()


def kernel(x, table):
    idx = x.reshape(-1)
    tab_lin = jnp.pad(table, ((0, 0), (0, LANES - D))).reshape(4 * V, D)
    out4 = _gather(idx, tab_lin)
    out5 = out4.reshape(T, 4, NW, 8, LANES)
    return out5.transpose(2, 4, 0, 1, 3).reshape(B, T, D)
